# baseline (speedup 1.0000x reference)
import functools

import jax
import jax.numpy as jnp
from jax import lax
from jax.experimental import pallas as pl
from jax.experimental.pallas import tpu as pltpu
from jax.experimental.pallas import tpu_sc as plsc

VOCAB = 1000000
D = 64
B = 16384
P = 20
N = 20
S = P + N

NC = 2
NS = 16
NW = NC * NS
BPW = B // NW

C = 16
NCHUNK = BPW // C
IDX_ROWS = C * S // 128

_mesh = plsc.VectorSubcoreMesh(core_axis_name="c", subcore_axis_name="s")


def _sc_logits_body(x_hbm, samp_hbm, emb_hbm, w_hbm, pos_hbm, neg_hbm,
                    xidx_v, sidx_v, emb_v, rows_v, pos_v, neg_v, sem1, sem2):
    wid = lax.axis_index("s") * NC + lax.axis_index("c")
    base = wid * BPW

    def chunk_body(i, _):
        b0 = base + i * C
        pltpu.sync_copy(x_hbm.at[pl.ds(b0, C)], xidx_v)
        pltpu.sync_copy(samp_hbm.at[pl.ds(b0 * S, C * S)], sidx_v)
        cp_e = pltpu.async_copy(emb_hbm.at[xidx_v], emb_v, sem1)
        for k in range(IDX_ROWS):
            pltpu.async_copy(
                w_hbm.at[sidx_v.at[pl.ds(k * 128, 128)]],
                rows_v.at[pl.ds(k * 128, 128)], sem2
            )
        cp_e.wait()
        for k in range(IDX_ROWS):
            pltpu.make_async_copy(
                w_hbm.at[sidx_v.at[pl.ds(k * 128, 128)]],
                rows_v.at[pl.ds(k * 128, 128)], sem2
            ).wait()

        lane = lax.iota(jnp.int32, 16)
        last_lane = lane == 15

        def b_body(b, _):
            e0 = emb_v[b, pl.ds(0, 16)]
            e1 = emb_v[b, pl.ds(16, 16)]
            e2 = emb_v[b, pl.ds(32, 16)]
            e3 = emb_v[b, pl.ds(48, 16)]
            for si in range(S):
                r = b * S + si
                acc = (rows_v[r, pl.ds(0, 16)] * e0
                       + rows_v[r, pl.ds(16, 16)] * e1
                       + rows_v[r, pl.ds(32, 16)] * e2
                       + rows_v[r, pl.ds(48, 16)] * e3)
                cum = plsc.cumsum(acc)
                if si < P:
                    tgt, pos = pos_v, b * P + si
                else:
                    tgt, pos = neg_v, b * N + (si - P)
                plsc.store_scatter(
                    tgt, [jnp.broadcast_to(pos, (16,))], cum, mask=last_lane
                )
            return 0

        lax.fori_loop(0, C, b_body, 0)
        pltpu.sync_copy(pos_v, pos_hbm.at[pl.ds(b0 * P, C * P)])
        pltpu.sync_copy(neg_v, neg_hbm.at[pl.ds(b0 * N, C * N)])
        return 0

    lax.fori_loop(0, NCHUNK, chunk_body, 0)


def _make_sc_logits(interpret=False):
    return pl.kernel(
        _sc_logits_body,
        out_type=(
            jax.ShapeDtypeStruct((B * P,), jnp.float32),
            jax.ShapeDtypeStruct((B * N,), jnp.float32),
        ),
        mesh=_mesh,
        compiler_params=pltpu.CompilerParams(
            needs_layout_passes=False, use_tc_tiling_on_sc=False),
        interpret=interpret,
        scratch_types=[
            pltpu.VMEM((C,), jnp.int32),
            pltpu.VMEM((C * S,), jnp.int32),
            pltpu.VMEM((C, D), jnp.float32),
            pltpu.VMEM((C * S, D), jnp.float32),
            pltpu.VMEM((C * P,), jnp.float32),
            pltpu.VMEM((C * N,), jnp.float32),
            pltpu.SemaphoreType.DMA,
            pltpu.SemaphoreType.DMA,
        ],
    )


_sc_logits = _make_sc_logits()


def _loss_body(pos_ref, neg_ref, out_ref):
    zp = pos_ref[...]
    zn = neg_ref[...]
    sp = 1.0 / (1.0 + jnp.exp(-zp))
    sn = 1.0 / (1.0 + jnp.exp(-zn))
    pos_loss = -jnp.sum(jnp.log(sp)) * (1.0 / (B * P))
    neg_loss = -jnp.sum(jnp.log(1.0 - sn + 1e-3)) * (1.0 / (B * N))
    out_ref[0, 0] = pos_loss + neg_loss


_loss = pl.pallas_call(
    _loss_body,
    out_shape=jax.ShapeDtypeStruct((1, 1), jnp.float32),
    out_specs=pl.BlockSpec(memory_space=pltpu.SMEM),
)


def kernel(x, positive_samples, negative_samples, emb_table, out_weight):
    x32 = x.astype(jnp.int32)
    samp = jnp.concatenate(
        [positive_samples.astype(jnp.int32), negative_samples.astype(jnp.int32)],
        axis=1,
    ).reshape(B * S)
    pos_logits, neg_logits = _sc_logits(x32, samp, emb_table, out_weight)
    out = _loss(pos_logits.reshape(B * P // 128, 128),
                neg_logits.reshape(B * N // 128, 128))
    return out[0, 0]
